# baseline (speedup 1.0000x reference)
import functools

import jax
import jax.numpy as jnp
from jax import lax
from jax.experimental import pallas as pl
from jax.experimental.pallas import tpu as pltpu
from jax.experimental.pallas import tpu_sc as plsc

_NC, _NS = 2, 16
_NW = _NC * _NS
_C = 200
_NB = 4


@jax.jit
def _lookup(table, x):
    V, D = table.shape
    N, S = x.shape
    B = N * S
    b_per_w = B // _NW
    n_chunks = b_per_w // _C
    idx = jnp.transpose(x).reshape(B)
    mesh = plsc.VectorSubcoreMesh(
        core_axis_name="c", subcore_axis_name="s",
        num_cores=_NC, num_subcores=_NS,
    )

    @functools.partial(
        pl.kernel,
        mesh=mesh,
        out_type=jax.ShapeDtypeStruct((B, D), jnp.float32),
        compiler_params=pltpu.CompilerParams(use_tc_tiling_on_sc=True),
        scratch_types=[
            pltpu.VMEM((b_per_w,), jnp.int32),
            pltpu.VMEM((_C, D), jnp.float32),
            pltpu.VMEM((_C, D), jnp.float32),
            pltpu.VMEM((_C, D), jnp.float32),
            pltpu.VMEM((_C, D), jnp.float32),
            pltpu.SemaphoreType.DMA,
            pltpu.SemaphoreType.DMA,
            pltpu.SemaphoreType.DMA,
            pltpu.SemaphoreType.DMA,
            pltpu.SemaphoreType.DMA,
            pltpu.SemaphoreType.DMA,
            pltpu.SemaphoreType.DMA,
            pltpu.SemaphoreType.DMA,
        ],
    )
    def k(table_hbm, idx_hbm, out_hbm, idx_v, rows0, rows1, rows2, rows3,
          g0, g1, g2, g3, s0, s1, s2, s3):
        wid = lax.axis_index("s") * _NC + lax.axis_index("c")
        base = wid * b_per_w
        rows = (rows0, rows1, rows2, rows3)
        gsem = (g0, g1, g2, g3)
        ssem = (s0, s1, s2, s3)
        pltpu.sync_copy(idx_hbm.at[pl.ds(base, b_per_w)], idx_v)

        def gather(c, b):
            pltpu.async_copy(
                table_hbm.at[idx_v.at[pl.ds(c * _C, _C)]], rows[b], gsem[b]
            )

        def gwait(b):
            pltpu.make_async_copy(
                table_hbm.at[idx_v.at[pl.ds(0, _C)]], rows[b], gsem[b]
            ).wait()

        def store(c, b):
            pltpu.async_copy(
                rows[b], out_hbm.at[pl.ds(base + c * _C, _C)], ssem[b]
            )

        def swait(b):
            pltpu.make_async_copy(
                rows[b], out_hbm.at[pl.ds(base, _C)], ssem[b]
            ).wait()

        gather(0, 0)
        gather(1, 1)
        for c in range(2):
            gwait(c)
            store(c, c)
            gather(c + 2, c + 2)

        @pl.loop(2, n_chunks - 2, step=4)
        def _(c):
            for j in range(4):
                b = (2 + j) % _NB
                gwait(b)
                store(c + j, b)
                b2 = (b + 2) % _NB
                swait(b2)
                gather(c + j + 2, b2)

        for t in range(2):
            b = (n_chunks - 2 + t) % _NB
            gwait(b)
            store(n_chunks - 2 + t, b)
        for b in range(_NB):
            swait(b)

    out = k(table, idx)
    return out.reshape(S, N, D).transpose(1, 0, 2)


def kernel(x, embedding_matrix):
    return _lookup(embedding_matrix, x.astype(jnp.int32))


# speedup vs baseline: 1.0007x; 1.0007x over previous
import functools

import jax
import jax.numpy as jnp
from jax import lax
from jax.experimental import pallas as pl
from jax.experimental.pallas import tpu as pltpu
from jax.experimental.pallas import tpu_sc as plsc

_NC, _NS = 2, 16
_JB = 1
_IB = 32


@jax.jit
def _lookup(table, x):
    V, D = table.shape
    N, S = x.shape
    B = N * S
    jpw = S // _JB
    ipw = N // _IB
    xt = jnp.transpose(x)
    mesh = plsc.VectorSubcoreMesh(
        core_axis_name="c", subcore_axis_name="s",
        num_cores=_NC, num_subcores=_NS,
    )

    @functools.partial(
        pl.kernel,
        mesh=mesh,
        out_type=jax.ShapeDtypeStruct((B, D), jnp.float32),
        compiler_params=pltpu.CompilerParams(use_tc_tiling_on_sc=True),
        scratch_types=[
            pltpu.VMEM((jpw, ipw), jnp.int32),
            pltpu.VMEM((ipw, D), jnp.float32),
            pltpu.VMEM((ipw, D), jnp.float32),
            pltpu.SemaphoreType.DMA,
            pltpu.SemaphoreType.DMA,
            pltpu.SemaphoreType.DMA,
            pltpu.SemaphoreType.DMA,
        ],
    )
    def k(table_hbm, xt_hbm, out_hbm, idx_v, rows0, rows1, g0, g1, s0, s1):
        wid = lax.axis_index("s") * _NC + lax.axis_index("c")
        i0 = pl.multiple_of(wid * ipw, ipw)
        rows = (rows0, rows1)
        gsem = (g0, g1)
        ssem = (s0, s1)
        pltpu.sync_copy(xt_hbm.at[:, pl.ds(i0, ipw)], idx_v)

        def gather(j, b):
            pltpu.async_copy(
                table_hbm.at[idx_v.at[j]], rows[b], gsem[b]
            )

        def gwait(b):
            pltpu.make_async_copy(
                table_hbm.at[idx_v.at[0]], rows[b], gsem[b]
            ).wait()

        def store(j, b):
            off = pl.multiple_of(j * N + i0, ipw)
            pltpu.async_copy(
                rows[b], out_hbm.at[pl.ds(off, ipw)], ssem[b]
            )

        def swait(b):
            pltpu.make_async_copy(
                rows[b], out_hbm.at[pl.ds(i0, ipw)], ssem[b]
            ).wait()

        gather(0, 0)
        gather(1, 1)

        @pl.loop(0, jpw - 3, step=2)
        def _(j):
            for t in range(2):
                gwait(t)
                store(j + t, t)
                swait(t)
                gather(j + t + 2, t)

        for t in range(2):
            gwait(t)
            store(jpw - 2 + t, t)
        swait(0)
        swait(1)

    out = k(table, xt)
    return out.reshape(S, N, D).transpose(1, 0, 2)


def kernel(x, embedding_matrix):
    return _lookup(embedding_matrix, x.astype(jnp.int32))
